# Initial kernel scaffold; baseline (speedup 1.0000x reference)
import jax
import jax.numpy as jnp
from jax.experimental import pallas as pl


def kernel(input):
    raise NotImplementedError("write your pallas kernel here")


# speedup vs baseline: 11.0377x; 11.0377x over previous
import numpy as np
import jax
import jax.numpy as jnp
from jax.experimental import pallas as pl

_NE = 16
_NT = 16384
_BT = 256
_TP = 12
_M32 = 0xFFFFFFFF


def _tf_scalar(k0, k1, x0, x1):
    ks = (k0, k1, k0 ^ k1 ^ 0x1BD11BDA)
    rot = ((13, 15, 26, 6), (17, 29, 16, 24))
    x0 = (x0 + k0) & _M32
    x1 = (x1 + k1) & _M32
    for i in range(5):
        for r in rot[i % 2]:
            x0 = (x0 + x1) & _M32
            x1 = ((x1 << r) | (x1 >> (32 - r))) & _M32
            x1 ^= x0
        x0 = (x0 + ks[(i + 1) % 3]) & _M32
        x1 = (x1 + ks[(i + 2) % 3] + i + 1) & _M32
    return x0, x1


_K1 = _tf_scalar(0, 42, 0, 0)
_K2 = _tf_scalar(0, 42, 0, 1)
_K3 = _tf_scalar(0, 42, 0, 2)
_PSUB = []
_rng = _K3
for _ in range(_TP):
    _PSUB.append(_tf_scalar(_rng[0], _rng[1], 0, 1))
    _rng = _tf_scalar(_rng[0], _rng[1], 0, 0)

_TINY = float(np.finfo(np.float32).tiny)


def _tf_fold(key, ctr):
    k0, k1 = key
    ks = (k0, k1, k0 ^ k1 ^ 0x1BD11BDA)
    rot = ((13, 15, 26, 6), (17, 29, 16, 24))
    x0 = jnp.full(ctr.shape, jnp.uint32(k0), dtype=jnp.uint32)
    x1 = ctr + jnp.uint32(k1)
    for i in range(5):
        for r in rot[i % 2]:
            x0 = x0 + x1
            x1 = (x1 << jnp.uint32(r)) | (x1 >> jnp.uint32(32 - r))
            x1 = x1 ^ x0
        x0 = x0 + jnp.uint32(ks[(i + 1) % 3])
        x1 = x1 + jnp.uint32((ks[(i + 2) % 3] + i + 1) & _M32)
    return x0 ^ x1


def _u01(bits):
    f = jax.lax.bitcast_convert_type(
        (bits >> jnp.uint32(9)) | jnp.uint32(0x3F800000), jnp.float32)
    return f - jnp.float32(1.0)


def _body(logp_ref, out_ref):
    b = pl.program_id(0)
    sub_i = jax.lax.broadcasted_iota(jnp.int32, (_NE, _BT), 0)
    sub_u = sub_i.astype(jnp.uint32)
    lane_u = jax.lax.broadcasted_iota(jnp.uint32, (_NE, _BT), 1)
    tok = (b * _BT).astype(jnp.uint32) + lane_u
    ctr16 = tok * jnp.uint32(_NE) + sub_u
    logp = logp_ref[...]

    rm = _u01(_tf_fold(_K1, ctr16))

    base_g = tok * jnp.uint32(_NE * _NE)
    sampled = jnp.zeros((_NE, _BT), dtype=jnp.int32)
    for j in range(_NE):
        ctr = base_g + (jnp.uint32(_NE * j) + sub_u)
        u = _u01(_tf_fold(_K2, ctr))
        u = jnp.maximum(jnp.float32(_TINY), u + jnp.float32(_TINY))
        g = -jnp.log(-jnp.log(u)) + logp
        gmax = jnp.max(g, axis=0, keepdims=True)
        cand = jnp.where(g == gmax, sub_i, jnp.int32(_NE))
        sj = jnp.min(cand, axis=0, keepdims=True)
        sampled = jnp.where(sub_i == j, jnp.broadcast_to(sj, (_NE, _BT)), sampled)

    rates = jnp.zeros((_NE, _BT), dtype=jnp.float32)
    for e in range(_NE):
        rme = jnp.broadcast_to(rm[e:e + 1, :], (_NE, _BT))
        rates = jnp.where(sampled == e, rme, rates)

    neg_lam = -rates
    k = jnp.zeros((_NE, _BT), dtype=jnp.int32)
    log_prod = jnp.zeros((_NE, _BT), dtype=jnp.float32)
    for t in range(_TP):
        k = jnp.where(log_prod > neg_lam, k + 1, k)
        u = _u01(_tf_fold(_PSUB[t], ctr16))
        log_prod = log_prod + jnp.log(u)
    pois = jnp.where(rates == jnp.float32(0.0), jnp.float32(0.0),
                     (k - 1).astype(jnp.float32))

    v = jnp.zeros((_NE, _BT), dtype=jnp.float32)
    for j in range(_NE):
        sj = jnp.broadcast_to(sampled[j:j + 1, :], (_NE, _BT))
        pj = jnp.broadcast_to(pois[j:j + 1, :], (_NE, _BT))
        v = jnp.where(sub_i == sj, pj, v)
    vmax = jnp.max(v, axis=0, keepdims=True)
    cand = jnp.where(v == vmax, sub_i, jnp.int32(_NE))
    amax = jnp.min(cand, axis=0, keepdims=True)
    onehot = (sub_i == jnp.broadcast_to(amax, (_NE, _BT))).astype(jnp.float32)

    @pl.when(b == 0)
    def _init():
        out_ref[...] = jnp.zeros_like(out_ref)

    out_ref[...] += onehot


def kernel(input):
    num_tokens = input.shape[0]
    assert num_tokens == _NT and num_tokens % _NE == 0
    exponents = jnp.power(jnp.arange(1, _NE + 1, dtype=jnp.float32), -1.8)
    p = exponents / jnp.sum(exponents)
    logp_t = jnp.broadcast_to(jnp.log(p)[:, None], (_NE, _BT))

    hist_t = pl.pallas_call(
        _body,
        grid=(_NT // _BT,),
        in_specs=[pl.BlockSpec((_NE, _BT), lambda b: (0, 0))],
        out_specs=pl.BlockSpec((_NE, _BT), lambda b: (0, 0)),
        out_shape=jax.ShapeDtypeStruct((_NE, _BT), jnp.float32),
    )(logp_t)

    hist = jnp.sum(hist_t, axis=1)
    idx = jax.random.permutation(jax.random.key(43), _NE)
    return hist[idx]


# speedup vs baseline: 15.6514x; 1.4180x over previous
from functools import partial

import numpy as np
import jax
import jax.numpy as jnp
from jax.experimental import pallas as pl
from jax.experimental.shard_map import shard_map
from jax.sharding import PartitionSpec as P

_NE = 16
_NT = 16384
_BT = 256
_TP = 9
_M32 = 0xFFFFFFFF


def _tf_scalar(k0, k1, x0, x1):
    ks = (k0, k1, k0 ^ k1 ^ 0x1BD11BDA)
    rot = ((13, 15, 26, 6), (17, 29, 16, 24))
    x0 = (x0 + k0) & _M32
    x1 = (x1 + k1) & _M32
    for i in range(5):
        for r in rot[i % 2]:
            x0 = (x0 + x1) & _M32
            x1 = ((x1 << r) | (x1 >> (32 - r))) & _M32
            x1 ^= x0
        x0 = (x0 + ks[(i + 1) % 3]) & _M32
        x1 = (x1 + ks[(i + 2) % 3] + i + 1) & _M32
    return x0, x1


_K1 = _tf_scalar(0, 42, 0, 0)
_K2 = _tf_scalar(0, 42, 0, 1)
_K3 = _tf_scalar(0, 42, 0, 2)
_PSUB = []
_rng = _K3
for _ in range(_TP):
    _PSUB.append(_tf_scalar(_rng[0], _rng[1], 0, 1))
    _rng = _tf_scalar(_rng[0], _rng[1], 0, 0)

_TINY = float(np.finfo(np.float32).tiny)


def _tf_fold(key, ctr):
    k0, k1 = key
    ks = (k0, k1, k0 ^ k1 ^ 0x1BD11BDA)
    rot = ((13, 15, 26, 6), (17, 29, 16, 24))
    x0 = jnp.full(ctr.shape, jnp.uint32(k0), dtype=jnp.uint32)
    x1 = ctr + jnp.uint32(k1)
    for i in range(5):
        for r in rot[i % 2]:
            x0 = x0 + x1
            x1 = (x1 << jnp.uint32(r)) | (x1 >> jnp.uint32(32 - r))
            x1 = x1 ^ x0
        x0 = x0 + jnp.uint32(ks[(i + 1) % 3])
        x1 = x1 + jnp.uint32((ks[(i + 2) % 3] + i + 1) & _M32)
    return x0 ^ x1


def _u01(bits):
    f = jax.lax.bitcast_convert_type(
        (bits >> jnp.uint32(9)) | jnp.uint32(0x3F800000), jnp.float32)
    return f - jnp.float32(1.0)


def _body(logp_ref, base_ref, out_ref):
    b = pl.program_id(0)
    sub_i = jax.lax.broadcasted_iota(jnp.int32, (_NE, _BT), 0)
    sub_u = sub_i.astype(jnp.uint32)
    lane_u = jax.lax.broadcasted_iota(jnp.uint32, (_NE, _BT), 1)
    tok = base_ref[...] + (b * _BT).astype(jnp.uint32) + lane_u
    ctr16 = tok * jnp.uint32(_NE) + sub_u
    logp = logp_ref[...]

    rm = _u01(_tf_fold(_K1, ctr16))

    base_g = tok * jnp.uint32(_NE * _NE)
    sampled = jnp.zeros((_NE, _BT), dtype=jnp.int32)
    for j in range(_NE):
        ctr = base_g + (jnp.uint32(_NE * j) + sub_u)
        u = _u01(_tf_fold(_K2, ctr))
        u = jnp.maximum(jnp.float32(_TINY), u + jnp.float32(_TINY))
        g = -jnp.log(-jnp.log(u)) + logp
        gmax = jnp.max(g, axis=0, keepdims=True)
        cand = jnp.where(g == gmax, sub_i, jnp.int32(_NE))
        sj = jnp.min(cand, axis=0, keepdims=True)
        sampled = jnp.where(sub_i == j, jnp.broadcast_to(sj, (_NE, _BT)), sampled)

    rates = jnp.zeros((_NE, _BT), dtype=jnp.float32)
    for e in range(_NE):
        rme = jnp.broadcast_to(rm[e:e + 1, :], (_NE, _BT))
        rates = jnp.where(sampled == e, rme, rates)

    neg_lam = -rates
    k = jnp.zeros((_NE, _BT), dtype=jnp.int32)
    log_prod = jnp.zeros((_NE, _BT), dtype=jnp.float32)
    for t in range(_TP):
        k = jnp.where(log_prod > neg_lam, k + 1, k)
        u = _u01(_tf_fold(_PSUB[t], ctr16))
        log_prod = log_prod + jnp.log(u)
    pois = jnp.where(rates == jnp.float32(0.0), jnp.float32(0.0),
                     (k - 1).astype(jnp.float32))

    v = jnp.zeros((_NE, _BT), dtype=jnp.float32)
    for j in range(_NE):
        sj = jnp.broadcast_to(sampled[j:j + 1, :], (_NE, _BT))
        pj = jnp.broadcast_to(pois[j:j + 1, :], (_NE, _BT))
        v = jnp.where(sub_i == sj, pj, v)
    vmax = jnp.max(v, axis=0, keepdims=True)
    cand = jnp.where(v == vmax, sub_i, jnp.int32(_NE))
    amax = jnp.min(cand, axis=0, keepdims=True)
    onehot = (sub_i == jnp.broadcast_to(amax, (_NE, _BT))).astype(jnp.float32)

    @pl.when(b == 0)
    def _init():
        out_ref[...] = jnp.zeros_like(out_ref)

    out_ref[...] += onehot


def _run_shard(logp_t, base_t, tokens_per_shard):
    return pl.pallas_call(
        _body,
        grid=(tokens_per_shard // _BT,),
        in_specs=[pl.BlockSpec((_NE, _BT), lambda b: (0, 0)),
                  pl.BlockSpec((_NE, _BT), lambda b: (0, 0))],
        out_specs=pl.BlockSpec((_NE, _BT), lambda b: (0, 0)),
        out_shape=jax.ShapeDtypeStruct((_NE, _BT), jnp.float32),
    )(logp_t, base_t)


def kernel(input):
    num_tokens = input.shape[0]
    assert num_tokens == _NT and num_tokens % _NE == 0
    exponents = jnp.power(jnp.arange(1, _NE + 1, dtype=jnp.float32), -1.8)
    p = exponents / jnp.sum(exponents)
    logp_col = jnp.log(p)[:, None]

    nd = min(jax.device_count(), 2)
    tokens_per_shard = _NT // nd
    logp_t = jnp.broadcast_to(logp_col[None], (nd, _NE, _BT))
    base_t = jnp.broadcast_to(
        (jnp.arange(nd, dtype=jnp.uint32) * np.uint32(tokens_per_shard))[:, None, None],
        (nd, _NE, _BT))

    mesh = jax.make_mesh((nd,), ("x",))
    sh = jax.sharding.NamedSharding(mesh, P("x"))
    logp_t = jax.reshard(logp_t, sh)
    base_t = jax.reshard(base_t, sh)
    run = shard_map(
        lambda lp, bt: _run_shard(lp[0], bt[0], tokens_per_shard)[None],
        mesh=mesh, in_specs=(P("x"), P("x")), out_specs=P("x"),
        check_rep=False)
    hist_t = run(logp_t, base_t)

    hist = jnp.sum(hist_t, axis=(0, 2))
    idx = jax.random.permutation(jax.random.key(43), _NE)
    return hist[idx]


# speedup vs baseline: 18.2620x; 1.1668x over previous
from functools import partial

import numpy as np
import jax
import jax.numpy as jnp
from jax.experimental import pallas as pl
from jax.experimental.shard_map import shard_map
from jax.sharding import PartitionSpec as P

_NE = 16
_NT = 16384
_BT = 512
_TP = 9
_M32 = 0xFFFFFFFF


def _tf_scalar(k0, k1, x0, x1):
    ks = (k0, k1, k0 ^ k1 ^ 0x1BD11BDA)
    rot = ((13, 15, 26, 6), (17, 29, 16, 24))
    x0 = (x0 + k0) & _M32
    x1 = (x1 + k1) & _M32
    for i in range(5):
        for r in rot[i % 2]:
            x0 = (x0 + x1) & _M32
            x1 = ((x1 << r) | (x1 >> (32 - r))) & _M32
            x1 ^= x0
        x0 = (x0 + ks[(i + 1) % 3]) & _M32
        x1 = (x1 + ks[(i + 2) % 3] + i + 1) & _M32
    return x0, x1


_K1 = _tf_scalar(0, 42, 0, 0)
_K2 = _tf_scalar(0, 42, 0, 1)
_K3 = _tf_scalar(0, 42, 0, 2)
_PSUB = []
_rng = _K3
for _ in range(_TP):
    _PSUB.append(_tf_scalar(_rng[0], _rng[1], 0, 1))
    _rng = _tf_scalar(_rng[0], _rng[1], 0, 0)

_TINY = float(np.finfo(np.float32).tiny)


def _tf_fold(key, ctr):
    k0, k1 = key
    ks = (k0, k1, k0 ^ k1 ^ 0x1BD11BDA)
    rot = ((13, 15, 26, 6), (17, 29, 16, 24))
    x0 = jnp.full(ctr.shape, jnp.uint32(k0), dtype=jnp.uint32)
    x1 = ctr + jnp.uint32(k1)
    for i in range(5):
        for r in rot[i % 2]:
            x0 = x0 + x1
            x1 = (x1 << jnp.uint32(r)) | (x1 >> jnp.uint32(32 - r))
            x1 = x1 ^ x0
        x0 = x0 + jnp.uint32(ks[(i + 1) % 3])
        x1 = x1 + jnp.uint32((ks[(i + 2) % 3] + i + 1) & _M32)
    return x0 ^ x1


def _u01(bits):
    f = jax.lax.bitcast_convert_type(
        (bits >> jnp.uint32(9)) | jnp.uint32(0x3F800000), jnp.float32)
    return f - jnp.float32(1.0)


def _body(logp_ref, base_ref, out_ref):
    b = pl.program_id(0)
    sub_i = jax.lax.broadcasted_iota(jnp.int32, (_NE, _BT), 0)
    sub_u = sub_i.astype(jnp.uint32)
    lane_u = jax.lax.broadcasted_iota(jnp.uint32, (_NE, _BT), 1)
    tok = base_ref[...] + (b * _BT).astype(jnp.uint32) + lane_u
    ctr16 = tok * jnp.uint32(_NE) + sub_u
    logp = logp_ref[...]

    rm = _u01(_tf_fold(_K1, ctr16))

    base_g = tok * jnp.uint32(_NE * _NE)
    sampled = jnp.zeros((_NE, _BT), dtype=jnp.int32)
    for j in range(_NE):
        ctr = base_g + (jnp.uint32(_NE * j) + sub_u)
        u = _u01(_tf_fold(_K2, ctr))
        u = jnp.maximum(jnp.float32(_TINY), u + jnp.float32(_TINY))
        g = -jnp.log(-jnp.log(u)) + logp
        gmax = jnp.max(g, axis=0, keepdims=True)
        cand = jnp.where(g == gmax, sub_i, jnp.int32(_NE))
        sj = jnp.min(cand, axis=0, keepdims=True)
        sampled = jnp.where(sub_i == j, jnp.broadcast_to(sj, (_NE, _BT)), sampled)

    rates = jnp.zeros((_NE, _BT), dtype=jnp.float32)
    for e in range(_NE):
        rme = jnp.broadcast_to(rm[e:e + 1, :], (_NE, _BT))
        rates = jnp.where(sampled == e, rme, rates)

    neg_lam = -rates
    k = jnp.zeros((_NE, _BT), dtype=jnp.int32)
    log_prod = jnp.zeros((_NE, _BT), dtype=jnp.float32)
    for t in range(_TP):
        k = jnp.where(log_prod > neg_lam, k + 1, k)
        u = _u01(_tf_fold(_PSUB[t], ctr16))
        log_prod = log_prod + jnp.log(u)
    pois = jnp.where(rates == jnp.float32(0.0), jnp.float32(0.0),
                     (k - 1).astype(jnp.float32))

    v = jnp.zeros((_NE, _BT), dtype=jnp.float32)
    for j in range(_NE):
        sj = jnp.broadcast_to(sampled[j:j + 1, :], (_NE, _BT))
        pj = jnp.broadcast_to(pois[j:j + 1, :], (_NE, _BT))
        v = jnp.where(sub_i == sj, pj, v)
    vmax = jnp.max(v, axis=0, keepdims=True)
    cand = jnp.where(v == vmax, sub_i, jnp.int32(_NE))
    amax = jnp.min(cand, axis=0, keepdims=True)
    onehot = (sub_i == jnp.broadcast_to(amax, (_NE, _BT))).astype(jnp.float32)

    @pl.when(b == 0)
    def _init():
        out_ref[...] = jnp.zeros_like(out_ref)

    out_ref[...] += onehot


def _run_shard(logp_t, base_t, tokens_per_shard):
    return pl.pallas_call(
        _body,
        grid=(tokens_per_shard // _BT,),
        in_specs=[pl.BlockSpec((_NE, _BT), lambda b: (0, 0)),
                  pl.BlockSpec((_NE, _BT), lambda b: (0, 0))],
        out_specs=pl.BlockSpec((_NE, _BT), lambda b: (0, 0)),
        out_shape=jax.ShapeDtypeStruct((_NE, _BT), jnp.float32),
    )(logp_t, base_t)


def kernel(input):
    num_tokens = input.shape[0]
    assert num_tokens == _NT and num_tokens % _NE == 0
    nd = min(jax.device_count(), 2)
    tokens_per_shard = _NT // nd

    def _shard_fn():
        exponents = jnp.power(jnp.arange(1, _NE + 1, dtype=jnp.float32), -1.8)
        p = exponents / jnp.sum(exponents)
        logp_t = jnp.broadcast_to(jnp.log(p)[:, None], (_NE, _BT))
        base = jax.lax.axis_index("x").astype(jnp.uint32) * np.uint32(tokens_per_shard)
        base_t = jnp.full((_NE, _BT), base, dtype=jnp.uint32)
        part = _run_shard(logp_t, base_t, tokens_per_shard)
        return jnp.sum(part, axis=1)[None]

    mesh = jax.make_mesh((nd,), ("x",))
    run = shard_map(_shard_fn, mesh=mesh, in_specs=(), out_specs=P("x"),
                    check_rep=False)
    hist = jnp.sum(run(), axis=0)
    idx = jax.random.permutation(jax.random.key(43), _NE)
    return hist[idx]
